# baseline (speedup 1.0000x reference)
import functools

import jax
import jax.numpy as jnp
from jax import lax
from jax.experimental import pallas as pl
from jax.experimental.pallas import tpu as pltpu
from jax.experimental.pallas import tpu_sc as plsc

VOCAB = 1000000
EMB = 32
NPOS = 512
PDIM = 32
B, L = 1024, 200

T = B * L
NC, NS = 2, 16
NW = NC * NS
TPW = T // NW
SEQ_PER_W = TPW // L
IPS = 128
CPT = 1280
SPC = CPT // IPS
NCHUNK = TPW // CPT


def _body(ids_hbm, word_hbm, pos_hbm, out_hbm, idx_v, rows_v, pos_v, sem):
    c = lax.axis_index("c")
    s = lax.axis_index("s")
    wid = s * NC + c
    base = wid * TPW

    pltpu.sync_copy(ids_hbm.at[wid], idx_v)

    pltpu.sync_copy(pos_hbm.at[pl.ds(0, L)], pos_v)

    def pos_seq(i, carry):
        tok0 = base + i * L
        pltpu.sync_copy(pos_v, out_hbm.at[pl.ds(tok0, L), 1])
        return carry

    lax.fori_loop(0, SEQ_PER_W, pos_seq, 0)

    def word_chunk(g, carry):
        tok0 = base + g * CPT
        cps = [
            pltpu.async_copy(
                word_hbm.at[idx_v.at[g * SPC + j]],
                rows_v.at[pl.ds(j * IPS, IPS)],
                sem,
            )
            for j in range(SPC)
        ]
        for cp in cps:
            cp.wait()
        pltpu.sync_copy(rows_v, out_hbm.at[pl.ds(tok0, CPT), 0])
        return carry

    lax.fori_loop(0, NCHUNK, word_chunk, 0)


@jax.jit
def kernel(input_ids, word_table, pos_table):
    ids2 = input_ids.astype(jnp.int32).reshape(NW, TPW // IPS, IPS)
    mesh = plsc.VectorSubcoreMesh(
        core_axis_name="c", subcore_axis_name="s", num_cores=NC, num_subcores=NS
    )
    out = pl.kernel(
        _body,
        out_type=jax.ShapeDtypeStruct((T, 2, EMB), jnp.float32),
        mesh=mesh,
        compiler_params=pltpu.CompilerParams(use_tc_tiling_on_sc=False),
        scratch_types=[
            pltpu.VMEM((TPW // IPS, IPS), jnp.int32),
            pltpu.VMEM((CPT, EMB), jnp.float32),
            pltpu.VMEM((L, PDIM), jnp.float32),
            pltpu.SemaphoreType.DMA,
        ],
    )(ids2, word_table, pos_table)
    return out.reshape(B, L, 2 * EMB)
